# baseline (speedup 1.0000x reference)
import functools

import jax
import jax.numpy as jnp
from jax import lax
from jax.experimental import pallas as pl
from jax.experimental.pallas import tpu as pltpu
from jax.experimental.pallas import tpu_sc as plsc

N_NODES = 10000
N_EDGES = 320000
D = 128
N_GRAPHS = 256

NC = 2
NS = 16
NW = NC * NS
CHUNK = 125
NCHUNK = N_EDGES // (NW * CHUNK)
BLK = 10
NBLK = NCHUNK // BLK
N_PAD = 10240
ROWS_PER_TILE = N_PAD // NS
ZCH = 32


def _sc_agg_kernel(x_hbm, edges_hbm, out_hbm,
                   isA, idA, isB, idB, r0, r1, zbuf, agg_sh,
                   sem_g0, sem_g1, sem_s, sem_ia, sem_ib, sem_d):
    c = lax.axis_index("c")
    s = lax.axis_index("s")
    row0 = s * ROWS_PER_TILE

    def _fetch_idx(blk, sref, dref, sem):
        pltpu.async_copy(edges_hbm.at[0, c, s, blk], sref, sem)
        pltpu.async_copy(edges_hbm.at[1, c, s, blk], dref, sem)

    def _wait_idx(blk, sref, dref, sem):
        pltpu.make_async_copy(edges_hbm.at[0, c, s, blk], sref, sem).wait()
        pltpu.make_async_copy(edges_hbm.at[1, c, s, blk], dref, sem).wait()

    _fetch_idx(0, isA, idA, sem_ia)

    zeros16 = jnp.zeros((16,), jnp.float32)

    def _zero_body(t, _):
        zbuf[t // 8, pl.ds((t % 8) * 16, 16)] = zeros16
        return 0

    lax.fori_loop(0, ZCH * 8, _zero_body, 0)
    for m in range(ROWS_PER_TILE // ZCH):
        pltpu.sync_copy(zbuf, agg_sh.at[pl.ds(row0 + m * ZCH, ZCH)])

    _wait_idx(0, isA, idA, sem_ia)
    pltpu.async_copy(x_hbm.at[isA.at[0, pl.ds(0, 64)]], r0.at[pl.ds(0, 64)], sem_g0)
    pltpu.async_copy(x_hbm.at[isA.at[0, pl.ds(64, 61)]], r0.at[pl.ds(64, 61)], sem_g0)
    pltpu.async_copy(x_hbm.at[isA.at[1, pl.ds(0, 64)]], r1.at[pl.ds(0, 64)], sem_g1)
    pltpu.async_copy(x_hbm.at[isA.at[1, pl.ds(64, 61)]], r1.at[pl.ds(64, 61)], sem_g1)
    plsc.subcore_barrier()

    bufs = (r0, r1)
    sem_g = (sem_g0, sem_g1)

    def _body(q, _):
        g0 = 2 * BLK * q
        for gl in range(2 * BLK):
            sref, dref = (isA, idA) if gl < BLK else (isB, idB)
            row = gl % BLK
            k = gl % 2

            if gl == 2:
                _fetch_idx(2 * q + 1, isB, idB, sem_ib)
            if gl == BLK + 2:
                @pl.when(g0 + 2 * BLK < NCHUNK)
                def _():
                    _fetch_idx(2 * q + 2, isA, idA, sem_ia)

            pltpu.make_async_copy(x_hbm.at[sref.at[row, pl.ds(0, 64)]],
                                  bufs[k].at[pl.ds(0, 64)], sem_g[k]).wait()
            pltpu.make_async_copy(x_hbm.at[sref.at[row, pl.ds(64, 61)]],
                                  bufs[k].at[pl.ds(64, 61)], sem_g[k]).wait()
            pltpu.async_copy(bufs[k], agg_sh.at[dref.at[row]], sem_s,
                             add=True).wait()

            if gl == BLK - 2:
                _wait_idx(2 * q + 1, isB, idB, sem_ib)
            if gl == 2 * BLK - 2:
                @pl.when(g0 + 2 * BLK < NCHUNK)
                def _():
                    _wait_idx(2 * q + 2, isA, idA, sem_ia)

            gl2 = gl + 2
            sref2 = isA if (gl2 < BLK or gl2 >= 2 * BLK) else isB
            row2 = gl2 % BLK

            @pl.when(g0 + gl2 < NCHUNK)
            def _():
                pltpu.async_copy(x_hbm.at[sref2.at[row2, pl.ds(0, 64)]],
                                 bufs[k].at[pl.ds(0, 64)], sem_g[k])
                pltpu.async_copy(x_hbm.at[sref2.at[row2, pl.ds(64, 61)]],
                                 bufs[k].at[pl.ds(64, 61)], sem_g[k])
        return 0

    lax.fori_loop(0, NBLK // 2, _body, 0)
    plsc.subcore_barrier()

    pltpu.async_copy(
        agg_sh.at[pl.ds(row0, ROWS_PER_TILE)],
        out_hbm.at[c, pl.ds(row0, ROWS_PER_TILE)],
        sem_d,
    ).wait()


@jax.jit
def _sc_aggregate(x, edges):
    run = pl.kernel(
        _sc_agg_kernel,
        mesh=plsc.VectorSubcoreMesh(core_axis_name="c", subcore_axis_name="s"),
        out_type=jax.ShapeDtypeStruct((NC, N_PAD, D), jnp.float32),
        scratch_types=[
            pltpu.VMEM((BLK, CHUNK), jnp.int32),
            pltpu.VMEM((BLK, CHUNK), jnp.int32),
            pltpu.VMEM((BLK, CHUNK), jnp.int32),
            pltpu.VMEM((BLK, CHUNK), jnp.int32),
            pltpu.VMEM((CHUNK, D), jnp.float32),
            pltpu.VMEM((CHUNK, D), jnp.float32),
            pltpu.VMEM((ZCH, D), jnp.float32),
            pltpu.VMEM_SHARED((N_PAD, D), jnp.float32),
        ] + [pltpu.SemaphoreType.DMA] * 6,
    )
    return run(x, edges)


RB = 1000
NTB = N_NODES // RB


def _tc_body(x_ref, agg_ref, batch_ref, w1_ref, b1_ref, w2_ref, b2_ref,
             w3_ref, b3_ref, gamma_ref, beta_ref, out_ref,
             h_scr, sums, sumsq, stats):
    p = pl.program_id(0)
    i = pl.program_id(1)

    @pl.when(p == 0)
    def _mlp_pass():
        hb = x_ref[...] + agg_ref[0] + agg_ref[1]
        hb = jnp.maximum(jnp.dot(hb, w1_ref[...],
                                 preferred_element_type=jnp.float32)
                         + b1_ref[...], 0.0)
        hb = jnp.maximum(jnp.dot(hb, w2_ref[...],
                                 preferred_element_type=jnp.float32)
                         + b2_ref[...], 0.0)
        hb = jnp.dot(hb, w3_ref[...],
                     preferred_element_type=jnp.float32) + b3_ref[...]
        hb = jnp.maximum(hb, 0.0)
        h_scr[pl.ds(i * RB, RB), :] = hb

        @pl.when(i == 0)
        def _():
            sums[...] = jnp.zeros((1, D), jnp.float32)
            sumsq[...] = jnp.zeros((1, D), jnp.float32)

        sums[...] += jnp.sum(hb, axis=0, keepdims=True)
        sumsq[...] += jnp.sum(hb * hb, axis=0, keepdims=True)

    @pl.when(p == 1)
    def _pool_pass():
        @pl.when(i == 0)
        def _():
            mean = sums[...] / N_NODES
            var = sumsq[...] / N_NODES - mean * mean
            stats[0:1, :] = mean
            stats[1:2, :] = jax.lax.rsqrt(var + 1e-5)
            out_ref[...] = jnp.zeros((N_GRAPHS, D), jnp.float32)

        hb = h_scr[pl.ds(i * RB, RB), :]
        hn = ((hb - stats[0:1, :]) * stats[1:2, :] * gamma_ref[...]
              + beta_ref[...])
        onehot = (batch_ref[...] ==
                  lax.broadcasted_iota(jnp.int32, (RB, N_GRAPHS), 1)
                  ).astype(jnp.float32)
        out_ref[...] += lax.dot_general(
            onehot, hn, (((0,), (0,)), ((), ())),
            preferred_element_type=jnp.float32)


def _tc_tail(x, agg, batch2d, W1, b1, W2, b2, W3, b3, gamma, beta):
    wspec = pl.BlockSpec((D, D), lambda p, i: (0, 0))
    bspec = pl.BlockSpec((1, D), lambda p, i: (0, 0))
    return pl.pallas_call(
        _tc_body,
        out_shape=jax.ShapeDtypeStruct((N_GRAPHS, D), jnp.float32),
        grid=(2, NTB),
        in_specs=[
            pl.BlockSpec((RB, D), lambda p, i: (i * (1 - p), 0)),
            pl.BlockSpec((NC, RB, D), lambda p, i: (0, i * (1 - p), 0)),
            pl.BlockSpec((RB, 1), lambda p, i: (i * p, 0)),
            wspec, bspec, wspec, bspec, wspec, bspec, bspec, bspec,
        ],
        out_specs=pl.BlockSpec((N_GRAPHS, D), lambda p, i: (0, 0)),
        scratch_shapes=[
            pltpu.VMEM((N_NODES, D), jnp.float32),
            pltpu.VMEM((1, D), jnp.float32),
            pltpu.VMEM((1, D), jnp.float32),
            pltpu.VMEM((2, D), jnp.float32),
        ],
    )(x, agg, batch2d, W1, b1, W2, b2, W3, b3, gamma, beta)


@jax.jit
def kernel(x, edge_index, batch, W1, b1, W2, b2, W3, b3, gamma, beta):
    edges = edge_index.astype(jnp.int32).reshape(2, NC, NS, NBLK, BLK, CHUNK)
    agg = _sc_aggregate(x, edges)
    batch2d = batch.astype(jnp.int32).reshape(N_NODES, 1)
    return _tc_tail(x, agg, batch2d,
                    W1, b1.reshape(1, D), W2, b2.reshape(1, D),
                    W3, b3.reshape(1, D), gamma.reshape(1, D),
                    beta.reshape(1, D))


# speedup vs baseline: 1.0507x; 1.0507x over previous
import functools

import jax
import jax.numpy as jnp
from jax import lax
from jax.experimental import pallas as pl
from jax.experimental.pallas import tpu as pltpu
from jax.experimental.pallas import tpu_sc as plsc

N_NODES = 10000
N_EDGES = 320000
D = 128
N_GRAPHS = 256

NC = 2
NS = 16
NW = NC * NS
CHUNK = 125
NCHUNK = N_EDGES // (NW * CHUNK)
BLK = 10
NBLK = NCHUNK // BLK
N_PAD = 10240
ROWS_PER_TILE = N_PAD // NS
ZCH = 32


def _sc_agg_kernel(x_hbm, edges_hbm, out_hbm,
                   isA, idA, isB, idB, r0, r1, zbuf, agg_sh,
                   sem_g0, sem_g1, sem_s, sem_ia, sem_ib, sem_d):
    c = lax.axis_index("c")
    s = lax.axis_index("s")
    row0 = s * ROWS_PER_TILE

    def _fetch_idx(blk, sref, dref, sem):
        pltpu.async_copy(edges_hbm.at[0, c, s, blk], sref, sem)
        pltpu.async_copy(edges_hbm.at[1, c, s, blk], dref, sem)

    def _wait_idx(blk, sref, dref, sem):
        pltpu.make_async_copy(edges_hbm.at[0, c, s, blk], sref, sem).wait()
        pltpu.make_async_copy(edges_hbm.at[1, c, s, blk], dref, sem).wait()

    _fetch_idx(0, isA, idA, sem_ia)

    zeros16 = jnp.zeros((16,), jnp.float32)

    def _zero_body(t, _):
        zbuf[t // 8, pl.ds((t % 8) * 16, 16)] = zeros16
        return 0

    lax.fori_loop(0, ZCH * 8, _zero_body, 0)
    for m in range(ROWS_PER_TILE // ZCH):
        pltpu.sync_copy(zbuf, agg_sh.at[pl.ds(row0 + m * ZCH, ZCH)])

    _wait_idx(0, isA, idA, sem_ia)
    pltpu.async_copy(x_hbm.at[isA.at[0, pl.ds(0, 64)]], r0.at[pl.ds(0, 64)], sem_g0)
    pltpu.async_copy(x_hbm.at[isA.at[0, pl.ds(64, 61)]], r0.at[pl.ds(64, 61)], sem_g0)
    pltpu.async_copy(x_hbm.at[isA.at[1, pl.ds(0, 64)]], r1.at[pl.ds(0, 64)], sem_g1)
    pltpu.async_copy(x_hbm.at[isA.at[1, pl.ds(64, 61)]], r1.at[pl.ds(64, 61)], sem_g1)
    plsc.subcore_barrier()

    bufs = (r0, r1)
    sem_g = (sem_g0, sem_g1)

    def _body(q, _):
        g0 = 2 * BLK * q
        for gl in range(2 * BLK):
            sref, dref = (isA, idA) if gl < BLK else (isB, idB)
            row = gl % BLK
            k = gl % 2

            if gl == 2:
                _fetch_idx(2 * q + 1, isB, idB, sem_ib)
            if gl == BLK + 2:
                @pl.when(g0 + 2 * BLK < NCHUNK)
                def _():
                    _fetch_idx(2 * q + 2, isA, idA, sem_ia)

            pltpu.make_async_copy(x_hbm.at[sref.at[row, pl.ds(0, 64)]],
                                  bufs[k].at[pl.ds(0, 64)], sem_g[k]).wait()
            pltpu.make_async_copy(x_hbm.at[sref.at[row, pl.ds(64, 61)]],
                                  bufs[k].at[pl.ds(64, 61)], sem_g[k]).wait()
            pltpu.async_copy(bufs[k], agg_sh.at[dref.at[row]], sem_s,
                             add=True).wait()

            if gl == BLK - 2:
                _wait_idx(2 * q + 1, isB, idB, sem_ib)
            if gl == 2 * BLK - 2:
                @pl.when(g0 + 2 * BLK < NCHUNK)
                def _():
                    _wait_idx(2 * q + 2, isA, idA, sem_ia)

            gl2 = gl + 2
            sref2 = isA if (gl2 < BLK or gl2 >= 2 * BLK) else isB
            row2 = gl2 % BLK

            @pl.when(g0 + gl2 < NCHUNK)
            def _():
                pltpu.async_copy(x_hbm.at[sref2.at[row2, pl.ds(0, 64)]],
                                 bufs[k].at[pl.ds(0, 64)], sem_g[k])
                pltpu.async_copy(x_hbm.at[sref2.at[row2, pl.ds(64, 61)]],
                                 bufs[k].at[pl.ds(64, 61)], sem_g[k])
        return 0

    lax.fori_loop(0, NBLK // 2, _body, 0)
    plsc.subcore_barrier()

    pltpu.async_copy(
        agg_sh.at[pl.ds(row0, ROWS_PER_TILE)],
        out_hbm.at[c, pl.ds(row0, ROWS_PER_TILE)],
        sem_d,
    ).wait()


@jax.jit
def _sc_aggregate(x, edges):
    run = pl.kernel(
        _sc_agg_kernel,
        mesh=plsc.VectorSubcoreMesh(core_axis_name="c", subcore_axis_name="s"),
        out_type=jax.ShapeDtypeStruct((NC, N_PAD, D), jnp.float32),
        scratch_types=[
            pltpu.VMEM((BLK, CHUNK), jnp.int32),
            pltpu.VMEM((BLK, CHUNK), jnp.int32),
            pltpu.VMEM((BLK, CHUNK), jnp.int32),
            pltpu.VMEM((BLK, CHUNK), jnp.int32),
            pltpu.VMEM((CHUNK, D), jnp.float32),
            pltpu.VMEM((CHUNK, D), jnp.float32),
            pltpu.VMEM((ZCH, D), jnp.float32),
            pltpu.VMEM_SHARED((N_PAD, D), jnp.float32),
        ] + [pltpu.SemaphoreType.DMA] * 6,
    )
    return run(x, edges)


RB = 5000
NTB = N_NODES // RB


def _tc_body(x_ref, agg_ref, batch_ref, w1_ref, b1_ref, w2_ref, b2_ref,
             w3_ref, b3_ref, gamma_ref, beta_ref, out_ref,
             h_scr, sums, sumsq, stats):
    p = pl.program_id(0)
    i = pl.program_id(1)

    @pl.when(p == 0)
    def _mlp_pass():
        hb = x_ref[...] + agg_ref[0] + agg_ref[1]
        hb = jnp.maximum(jnp.dot(hb, w1_ref[...],
                                 preferred_element_type=jnp.float32)
                         + b1_ref[...], 0.0)
        hb = jnp.maximum(jnp.dot(hb, w2_ref[...],
                                 preferred_element_type=jnp.float32)
                         + b2_ref[...], 0.0)
        hb = jnp.dot(hb, w3_ref[...],
                     preferred_element_type=jnp.float32) + b3_ref[...]
        hb = jnp.maximum(hb, 0.0)
        h_scr[pl.ds(i * RB, RB), :] = hb

        @pl.when(i == 0)
        def _():
            sums[...] = jnp.zeros((1, D), jnp.float32)
            sumsq[...] = jnp.zeros((1, D), jnp.float32)

        sums[...] += jnp.sum(hb, axis=0, keepdims=True)
        sumsq[...] += jnp.sum(hb * hb, axis=0, keepdims=True)

    @pl.when(p == 1)
    def _pool_pass():
        @pl.when(i == 0)
        def _():
            mean = sums[...] / N_NODES
            var = sumsq[...] / N_NODES - mean * mean
            stats[0:1, :] = mean
            stats[1:2, :] = jax.lax.rsqrt(var + 1e-5)
            out_ref[...] = jnp.zeros((N_GRAPHS, D), jnp.float32)

        hb = h_scr[pl.ds(i * RB, RB), :]
        hn = ((hb - stats[0:1, :]) * stats[1:2, :] * gamma_ref[...]
              + beta_ref[...])
        onehot = (batch_ref[...] ==
                  lax.broadcasted_iota(jnp.int32, (RB, N_GRAPHS), 1)
                  ).astype(jnp.float32)
        out_ref[...] += lax.dot_general(
            onehot, hn, (((0,), (0,)), ((), ())),
            preferred_element_type=jnp.float32)


def _tc_tail(x, agg, batch2d, W1, b1, W2, b2, W3, b3, gamma, beta):
    wspec = pl.BlockSpec((D, D), lambda p, i: (0, 0))
    bspec = pl.BlockSpec((1, D), lambda p, i: (0, 0))
    return pl.pallas_call(
        _tc_body,
        out_shape=jax.ShapeDtypeStruct((N_GRAPHS, D), jnp.float32),
        grid=(2, NTB),
        in_specs=[
            pl.BlockSpec((RB, D), lambda p, i: (i * (1 - p), 0)),
            pl.BlockSpec((NC, RB, D), lambda p, i: (0, i * (1 - p), 0)),
            pl.BlockSpec((RB, 1), lambda p, i: (i * p, 0)),
            wspec, bspec, wspec, bspec, wspec, bspec, bspec, bspec,
        ],
        out_specs=pl.BlockSpec((N_GRAPHS, D), lambda p, i: (0, 0)),
        scratch_shapes=[
            pltpu.VMEM((N_NODES, D), jnp.float32),
            pltpu.VMEM((1, D), jnp.float32),
            pltpu.VMEM((1, D), jnp.float32),
            pltpu.VMEM((2, D), jnp.float32),
        ],
    )(x, agg, batch2d, W1, b1, W2, b2, W3, b3, gamma, beta)


@jax.jit
def kernel(x, edge_index, batch, W1, b1, W2, b2, W3, b3, gamma, beta):
    edges = edge_index.astype(jnp.int32).reshape(2, NC, NS, NBLK, BLK, CHUNK)
    agg = _sc_aggregate(x, edges)
    batch2d = batch.astype(jnp.int32).reshape(N_NODES, 1)
    return _tc_tail(x, agg, batch2d,
                    W1, b1.reshape(1, D), W2, b2.reshape(1, D),
                    W3, b3.reshape(1, D), gamma.reshape(1, D),
                    beta.reshape(1, D))
